# baseline (device time: 110245 ns/iter reference)
import jax
import jax.numpy as jnp
from jax import lax
from jax.experimental import pallas as pl
from jax.experimental.pallas import tpu as pltpu

N_DEV = 32


def kernel(x, dy):
    m, d_in = x.shape
    _, d_out = dy.shape
    chunk = d_in // N_DEV

    def body(x_ref, dy_ref, out_ref, p_ref, send_buf, recv_buf,
             send_sems, recv_sems):
        my = lax.axis_index("i")
        left = lax.rem(my + N_DEV - 1, N_DEV)
        right = lax.rem(my + 1, N_DEV)

        barrier_sem = pltpu.get_barrier_semaphore()
        for nbr in (left, right):
            pl.semaphore_signal(
                barrier_sem, inc=1,
                device_id=(nbr,), device_id_type=pl.DeviceIdType.MESH,
            )
        pl.semaphore_wait(barrier_sem, 2)

        p_ref[...] = lax.dot_general(
            x_ref[...], dy_ref[...],
            dimension_numbers=(((0,), (0,)), ((), ())),
            preferred_element_type=jnp.float32,
        )

        def p_chunk(c):
            return p_ref[pl.ds(c * chunk, chunk), :]

        send_buf[0, :, :] = p_chunk(lax.rem(my + N_DEV - 1, N_DEV))

        for s in range(N_DEV - 1):
            slot = s % 2
            rdma = pltpu.make_async_remote_copy(
                src_ref=send_buf.at[slot],
                dst_ref=recv_buf.at[slot],
                send_sem=send_sems.at[slot],
                recv_sem=recv_sems.at[slot],
                device_id=(right,),
                device_id_type=pl.DeviceIdType.MESH,
            )
            rdma.start()
            rdma.wait()
            c = lax.rem(my - s - 2 + 2 * N_DEV, N_DEV)
            acc = recv_buf[slot, :, :] + p_chunk(c)
            if s < N_DEV - 2:
                send_buf[(s + 1) % 2, :, :] = acc
            else:
                out_ref[...] = acc

    return pl.pallas_call(
        body,
        out_shape=jax.ShapeDtypeStruct((chunk, d_out), jnp.float32),
        in_specs=[
            pl.BlockSpec(memory_space=pltpu.VMEM),
            pl.BlockSpec(memory_space=pltpu.VMEM),
        ],
        out_specs=pl.BlockSpec(memory_space=pltpu.VMEM),
        scratch_shapes=[
            pltpu.VMEM((m, d_out), jnp.float32),
            pltpu.VMEM((2, chunk, d_out), jnp.float32),
            pltpu.VMEM((2, chunk, d_out), jnp.float32),
            pltpu.SemaphoreType.DMA((2,)),
            pltpu.SemaphoreType.DMA((2,)),
        ],
        compiler_params=pltpu.CompilerParams(collective_id=0),
    )(x, dy)


# device time: 71086 ns/iter; 1.5509x vs baseline; 1.5509x over previous
import functools

import jax
import jax.numpy as jnp
from jax import lax
from jax.experimental import pallas as pl
from jax.experimental.pallas import tpu as pltpu

N_DEV = 32


def kernel(x, dy):
    m, d_in = x.shape
    _, d_out = dy.shape
    chunk = d_in // N_DEV

    def body(x_ref, dy_ref, out_ref, p_ref, recv_buf, send_sems, recv_sems):
        my = lax.axis_index("i")

        barrier_sem = pltpu.get_barrier_semaphore()
        for o in range(1, N_DEV):
            peer = lax.rem(my + o, N_DEV)
            pl.semaphore_signal(
                barrier_sem, inc=1,
                device_id=(peer,), device_id_type=pl.DeviceIdType.MESH,
            )
        pl.semaphore_wait(barrier_sem, N_DEV - 1)

        p_ref[...] = lax.dot_general(
            x_ref[...], dy_ref[...],
            dimension_numbers=(((0,), (0,)), ((), ())),
            preferred_element_type=jnp.float32,
        )

        sends = []
        for o in range(1, N_DEV):
            t = lax.rem(my + o, N_DEV)
            rdma = pltpu.make_async_remote_copy(
                src_ref=p_ref.at[pl.ds(t * chunk, chunk), :],
                dst_ref=recv_buf.at[my],
                send_sem=send_sems.at[o],
                recv_sem=recv_sems.at[my],
                device_id=(t,),
                device_id_type=pl.DeviceIdType.MESH,
            )
            rdma.start()
            sends.append(rdma)

        acc = p_ref[pl.ds(my * chunk, chunk), :]
        for o in range(1, N_DEV):
            s = lax.rem(my - o + N_DEV, N_DEV)
            recv = pltpu.make_async_remote_copy(
                src_ref=p_ref.at[pl.ds(0, chunk), :],
                dst_ref=recv_buf.at[s],
                send_sem=send_sems.at[0],
                recv_sem=recv_sems.at[s],
                device_id=(s,),
                device_id_type=pl.DeviceIdType.MESH,
            )
            recv.wait_recv()
            acc = acc + recv_buf[s, :, :]
        out_ref[...] = acc

        for rdma in sends:
            rdma.wait_send()

        @functools.partial(pl.run_scoped, exit_sem=pltpu.SemaphoreType.REGULAR)
        def _(exit_sem):
            for o in range(1, N_DEV):
                peer = lax.rem(my + o, N_DEV)
                pl.semaphore_signal(
                    exit_sem, inc=1,
                    device_id=(peer,), device_id_type=pl.DeviceIdType.MESH,
                )
            pl.semaphore_wait(exit_sem, N_DEV - 1)

    return pl.pallas_call(
        body,
        out_shape=jax.ShapeDtypeStruct((chunk, d_out), jnp.float32),
        in_specs=[
            pl.BlockSpec(memory_space=pltpu.VMEM),
            pl.BlockSpec(memory_space=pltpu.VMEM),
        ],
        out_specs=pl.BlockSpec(memory_space=pltpu.VMEM),
        scratch_shapes=[
            pltpu.VMEM((m, d_out), jnp.float32),
            pltpu.VMEM((N_DEV, chunk, d_out), jnp.float32),
            pltpu.SemaphoreType.DMA((N_DEV,)),
            pltpu.SemaphoreType.DMA((N_DEV,)),
        ],
        compiler_params=pltpu.CompilerParams(collective_id=0),
    )(x, dy)


# device time: 41619 ns/iter; 2.6489x vs baseline; 1.7080x over previous
import functools

import jax
import jax.numpy as jnp
from jax import lax
from jax.experimental import pallas as pl
from jax.experimental.pallas import tpu as pltpu

N_DEV = 32
CHUNK = 16

STRIPES = (
    (0, 768, (1, 3, 0, 2, 4)),
    (768, 640, (3, 0, 1, 4, 2)),
    (1408, 640, (0, 1, 3, 4, 2)),
)
N_PHASES = 5
RECV_ROWS = tuple(256 >> j for j in range(N_PHASES))


def _rank_of_l(l):
    z = l // 8
    rem = l % 8
    y = rem // 2
    xx = rem % 2
    q = jnp.bitwise_xor(xx, y % 2)
    return 8 * z + 2 * y + q


def _l_of_rank(r):
    z = r // 8
    p = r % 8
    y = p // 2
    q = p % 2
    xx = jnp.bitwise_xor(q, y % 2)
    return 8 * z + 2 * y + xx


def kernel(x, dy):
    m, d_in = x.shape
    _, d_out = dy.shape

    def body(x_ref, dy_ref, out_ref, *scratch):
        p_ref = scratch[0]
        wbufs = scratch[1:4]
        rbufs = scratch[4:4 + 3 * N_PHASES]
        send_sems, recv_sems = scratch[4 + 3 * N_PHASES:]

        my = lax.axis_index("i")
        my_l = _l_of_rank(my)

        partner = [_rank_of_l(jnp.bitwise_xor(my_l, 1 << b)) for b in range(5)]

        barrier_sem = pltpu.get_barrier_semaphore()
        for pr in partner:
            pl.semaphore_signal(
                barrier_sem, inc=1,
                device_id=(pr,), device_id_type=pl.DeviceIdType.MESH,
            )
        pl.semaphore_wait(barrier_sem, 5)

        p_ref[...] = lax.dot_general(
            x_ref[...], dy_ref[...],
            dimension_numbers=(((0,), (0,)), ((), ())),
            preferred_element_type=jnp.float32,
        )

        def exchange(s, j, start_only):
            c0, w, order = STRIPES[s]
            rows = RECV_ROWS[j]
            rdma = pltpu.make_async_remote_copy(
                src_ref=wbufs[s].at[pl.ds(rows, rows), :],
                dst_ref=rbufs[N_PHASES * s + j],
                send_sem=send_sems.at[s, j],
                recv_sem=recv_sems.at[s, j],
                device_id=(partner[order[j]],),
                device_id_type=pl.DeviceIdType.MESH,
            )
            if start_only:
                rdma.start()
            return rdma

        sends = []
        for s in range(3):
            c0, w, order = STRIPES[s]
            for idx in range(32):
                mask = 0
                for j in range(N_PHASES):
                    if (idx >> (4 - j)) & 1:
                        mask |= 1 << order[j]
                c_rank = _rank_of_l(jnp.bitwise_xor(my_l, mask))
                wbufs[s][pl.ds(CHUNK * idx, CHUNK), :] = (
                    p_ref[pl.ds(c_rank * CHUNK, CHUNK), pl.ds(c0, w)]
                )
            sends.append(exchange(s, 0, True))

        for j in range(N_PHASES):
            rows = RECV_ROWS[j]
            for s in range(3):
                exchange(s, j, False).wait_recv()
                wbufs[s][pl.ds(0, rows), :] = (
                    wbufs[s][pl.ds(0, rows), :] + rbufs[N_PHASES * s + j][...]
                )
                if j + 1 < N_PHASES:
                    sends.append(exchange(s, j + 1, True))

        for s in range(3):
            c0, w, _ = STRIPES[s]
            out_ref[:, pl.ds(c0, w)] = wbufs[s][pl.ds(0, CHUNK), :]

        for rdma in sends:
            rdma.wait_send()

        @functools.partial(pl.run_scoped, exit_sem=pltpu.SemaphoreType.REGULAR)
        def _(exit_sem):
            for pr in partner:
                pl.semaphore_signal(
                    exit_sem, inc=1,
                    device_id=(pr,), device_id_type=pl.DeviceIdType.MESH,
                )
            pl.semaphore_wait(exit_sem, 5)

    scratch_shapes = [pltpu.VMEM((m, d_out), jnp.float32)]
    scratch_shapes += [
        pltpu.VMEM((m, w), jnp.float32) for (_, w, _) in STRIPES
    ]
    for (_, w, _) in STRIPES:
        scratch_shapes += [
            pltpu.VMEM((RECV_ROWS[j], w), jnp.float32) for j in range(N_PHASES)
        ]
    scratch_shapes += [
        pltpu.SemaphoreType.DMA((3, N_PHASES)),
        pltpu.SemaphoreType.DMA((3, N_PHASES)),
    ]

    return pl.pallas_call(
        body,
        out_shape=jax.ShapeDtypeStruct((CHUNK, d_out), jnp.float32),
        in_specs=[
            pl.BlockSpec(memory_space=pltpu.VMEM),
            pl.BlockSpec(memory_space=pltpu.VMEM),
        ],
        out_specs=pl.BlockSpec(memory_space=pltpu.VMEM),
        scratch_shapes=scratch_shapes,
        compiler_params=pltpu.CompilerParams(collective_id=0),
    )(x, dy)
